# Initial kernel scaffold; baseline (speedup 1.0000x reference)
import jax
import jax.numpy as jnp
from jax.experimental import pallas as pl


def kernel(in_feat, edge_index, params):
    raise NotImplementedError("write your pallas kernel here")


# speedup vs baseline: 10.6513x; 10.6513x over previous
import functools

import jax
import jax.numpy as jnp
from jax import lax
from jax.experimental import pallas as pl
from jax.experimental.pallas import tpu as pltpu
from jax.experimental.pallas import tpu_sc as plsc

N_NODES = 10000
LAT = 32
N_ITERS = 3
NC = 2
NS = 16
NW = NC * NS
B = 128
ROWS = 80
E_PAD = NW * ROWS * B
PAD_NODES = 10112
STRIPE = PAD_NODES // NS
DEG_PAD = 10240
DEG_STRIPE = DEG_PAD // NS


def _vmesh():
    return plsc.VectorSubcoreMesh(core_axis_name="c", subcore_axis_name="s")


_SC_PARAMS = pltpu.CompilerParams(use_tc_tiling_on_sc=False)


def _sc_agg(proc, src3, dst3, zeros_stripe):
    @functools.partial(
        pl.kernel,
        out_type=jax.ShapeDtypeStruct((NC, PAD_NODES, LAT), jnp.float32),
        mesh=_vmesh(),
        scratch_types=[
            pltpu.VMEM((ROWS, B), jnp.int32),
            pltpu.VMEM((ROWS, B), jnp.int32),
            pltpu.VMEM((B, LAT), jnp.float32),
            pltpu.VMEM_SHARED((PAD_NODES, LAT), jnp.float32),
            pltpu.SemaphoreType.DMA,
        ],
        compiler_params=_SC_PARAMS,
    )
    def k(proc_hbm, src_hbm, dst_hbm, z_hbm, out_hbm, srcv, dstv, rows, acc, gsem):
        c = lax.axis_index("c")
        s = lax.axis_index("s")
        w = c * NS + s
        pltpu.sync_copy(z_hbm, acc.at[pl.ds(s * STRIPE, STRIPE)])
        pltpu.sync_copy(src_hbm.at[w], srcv)
        pltpu.sync_copy(dst_hbm.at[w], dstv)
        plsc.subcore_barrier()

        def body(j, carry):
            pltpu.async_copy(proc_hbm.at[srcv.at[j]], rows, gsem).wait()
            pltpu.sync_copy(rows, acc.at[dstv.at[j]], add=True)
            return carry

        lax.fori_loop(0, ROWS, body, 0)
        plsc.subcore_barrier()
        pltpu.sync_copy(acc.at[pl.ds(s * STRIPE, STRIPE)],
                        out_hbm.at[c].at[pl.ds(s * STRIPE, STRIPE)])

    return k(proc, src3, dst3, zeros_stripe)


def _sc_deg(dst3, zeros_deg, ones_row):
    @functools.partial(
        pl.kernel,
        out_type=jax.ShapeDtypeStruct((NC, DEG_PAD), jnp.float32),
        mesh=_vmesh(),
        scratch_types=[
            pltpu.VMEM((ROWS, B), jnp.int32),
            pltpu.VMEM((B,), jnp.float32),
            pltpu.VMEM_SHARED((DEG_PAD,), jnp.float32),
        ],
        compiler_params=_SC_PARAMS,
    )
    def k(dst_hbm, z_hbm, ones_hbm, out_hbm, dstv, onesv, acc):
        c = lax.axis_index("c")
        s = lax.axis_index("s")
        w = c * NS + s
        pltpu.sync_copy(z_hbm, acc.at[pl.ds(s * DEG_STRIPE, DEG_STRIPE)])
        pltpu.sync_copy(dst_hbm.at[w], dstv)
        pltpu.sync_copy(ones_hbm, onesv)
        plsc.subcore_barrier()

        def body(j, carry):
            pltpu.sync_copy(onesv, acc.at[dstv.at[j]], add=True)
            return carry

        lax.fori_loop(0, ROWS, body, 0)
        plsc.subcore_barrier()
        pltpu.sync_copy(acc.at[pl.ds(s * DEG_STRIPE, DEG_STRIPE)],
                        out_hbm.at[c].at[pl.ds(s * DEG_STRIPE, DEG_STRIPE)])

    return k(dst3, zeros_deg, ones_row)


def _lrelu(x):
    return jnp.where(x >= 0, x, 0.01 * x)


def _row2(v):
    return v.reshape(1, -1)


def _enc_call(x, p):
    def body(x_ref, w0, b0, w1, b1, w2, b2, g, bt, o_ref):
        h = _lrelu(x_ref[...] @ w0[...] + b0[...])
        h = _lrelu(h @ w1[...] + b1[...])
        h = h @ w2[...] + b2[...]
        mu = jnp.mean(h, axis=-1, keepdims=True)
        var = jnp.mean((h - mu) ** 2, axis=-1, keepdims=True)
        o_ref[...] = (h - mu) * lax.rsqrt(var + 1e-5) * g[...] + bt[...]

    return pl.pallas_call(
        body,
        out_shape=jax.ShapeDtypeStruct((N_NODES, LAT), jnp.float32),
    )(x, p['Win'], _row2(p['bin']), p['Wh'][0], _row2(p['bh'][0]),
      p['Wout'], _row2(p['bout']), _row2(p['gamma']), _row2(p['beta']))


def _proc_call(proc, S, deg, p):
    def body(x_ref, s_ref, d_ref, wa, wb, b0, w1, b1, w2, b2, g, bt, o_ref):
        x = x_ref[...]
        pe = (s_ref[0, :N_NODES, :] + s_ref[1, :N_NODES, :]
              + (d_ref[0, :N_NODES, :] + d_ref[1, :N_NODES, :]) * x)
        h = _lrelu(x @ wa[...] + pe @ wb[...] + b0[...])
        h = _lrelu(h @ w1[...] + b1[...])
        h = h @ w2[...] + b2[...]
        mu = jnp.mean(h, axis=-1, keepdims=True)
        var = jnp.mean((h - mu) ** 2, axis=-1, keepdims=True)
        o_ref[...] = (h - mu) * lax.rsqrt(var + 1e-5) * g[...] + bt[...] + x

    return pl.pallas_call(
        body,
        out_shape=jax.ShapeDtypeStruct((N_NODES, LAT), jnp.float32),
    )(proc, S, deg, p['Win'][:LAT], p['Win'][LAT:], _row2(p['bin']),
      p['Wh'][0], _row2(p['bh'][0]), p['Wout'], _row2(p['bout']),
      _row2(p['gamma']), _row2(p['beta']))


def _out_call(proc, p):
    def body(x_ref, w0, b0, w1, b1, w2, b2, o_ref):
        h = _lrelu(x_ref[...] @ w0[...] + b0[...])
        h = _lrelu(h @ w1[...] + b1[...])
        o_ref[...] = h @ w2[...] + b2[...]

    return pl.pallas_call(
        body,
        out_shape=jax.ShapeDtypeStruct((N_NODES, p['Wout'].shape[1]), jnp.float32),
    )(proc, p['Win'], _row2(p['bin']), p['Wh'][0], _row2(p['bh'][0]),
      p['Wout'], _row2(p['bout']))


def kernel(in_feat, edge_index, params):
    src = edge_index[0]
    dst = edge_index[1]
    npad = E_PAD - src.shape[0]
    src3 = jnp.concatenate([src, jnp.zeros((npad,), jnp.int32)]).reshape(NW, ROWS, B)
    dst3 = jnp.concatenate([dst, jnp.full((npad,), N_NODES, jnp.int32)]).reshape(NW, ROWS, B)
    zeros_s = jnp.zeros((STRIPE, LAT), jnp.float32)
    zeros_d = jnp.zeros((DEG_STRIPE,), jnp.float32)
    ones_r = jnp.ones((B,), jnp.float32)

    deg = _sc_deg(dst3, zeros_d, ones_r).reshape(NC, DEG_PAD, 1)
    proc = _enc_call(in_feat, params['enc'])
    for i in range(N_ITERS):
        S = _sc_agg(proc, src3, dst3, zeros_s)
        proc = _proc_call(proc, S, deg, params['proc'][i])
    return _out_call(proc, params['out'])


# speedup vs baseline: 13.0902x; 1.2290x over previous
import functools

import jax
import jax.numpy as jnp
from jax import lax
from jax.experimental import pallas as pl
from jax.experimental.pallas import tpu as pltpu
from jax.experimental.pallas import tpu_sc as plsc

N_NODES = 10000
LAT = 32
N_ITERS = 3
NC = 2
NS = 16
NW = NC * NS
B = 128
ROWS = 80
E_PAD = NW * ROWS * B
PAD_NODES = 10112
STRIPE = PAD_NODES // NS
DEG_PAD = 10240
DEG_STRIPE = DEG_PAD // NS


def _vmesh():
    return plsc.VectorSubcoreMesh(core_axis_name="c", subcore_axis_name="s")


_SC_PARAMS = pltpu.CompilerParams(use_tc_tiling_on_sc=False)


def _sc_agg(proc, src3, dst3, zeros_stripe):
    @functools.partial(
        pl.kernel,
        out_type=jax.ShapeDtypeStruct((NC, PAD_NODES, LAT), jnp.float32),
        mesh=_vmesh(),
        scratch_types=[
            pltpu.VMEM((ROWS, B), jnp.int32),
            pltpu.VMEM((ROWS, B), jnp.int32),
            pltpu.VMEM((B, LAT), jnp.float32),
            pltpu.VMEM((B, LAT), jnp.float32),
            pltpu.VMEM_SHARED((PAD_NODES, LAT), jnp.float32),
            pltpu.SemaphoreType.DMA,
            pltpu.SemaphoreType.DMA,
        ],
        compiler_params=_SC_PARAMS,
    )
    def k(proc_hbm, src_hbm, dst_hbm, z_hbm, out_hbm,
          srcv, dstv, rows0, rows1, acc, sem0, sem1):
        c = lax.axis_index("c")
        s = lax.axis_index("s")
        w = c * NS + s
        pltpu.sync_copy(z_hbm, acc.at[pl.ds(s * STRIPE, STRIPE)])
        pltpu.sync_copy(src_hbm.at[w], srcv)
        pltpu.sync_copy(dst_hbm.at[w], dstv)
        plsc.subcore_barrier()

        pltpu.async_copy(proc_hbm.at[srcv.at[0]], rows0, sem0)

        def body(jj, carry):
            j0 = 2 * jj
            pltpu.async_copy(proc_hbm.at[srcv.at[j0 + 1]], rows1, sem1)
            pltpu.make_async_copy(proc_hbm.at[srcv.at[j0]], rows0, sem0).wait()
            pltpu.sync_copy(rows0, acc.at[dstv.at[j0]], add=True)
            pltpu.async_copy(proc_hbm.at[srcv.at[j0 + 2]], rows0, sem0)
            pltpu.make_async_copy(proc_hbm.at[srcv.at[j0 + 1]], rows1, sem1).wait()
            pltpu.sync_copy(rows1, acc.at[dstv.at[j0 + 1]], add=True)
            return carry

        lax.fori_loop(0, ROWS // 2 - 1, body, 0)
        pltpu.async_copy(proc_hbm.at[srcv.at[ROWS - 1]], rows1, sem1)
        pltpu.make_async_copy(proc_hbm.at[srcv.at[ROWS - 2]], rows0, sem0).wait()
        pltpu.sync_copy(rows0, acc.at[dstv.at[ROWS - 2]], add=True)
        pltpu.make_async_copy(proc_hbm.at[srcv.at[ROWS - 1]], rows1, sem1).wait()
        pltpu.sync_copy(rows1, acc.at[dstv.at[ROWS - 1]], add=True)
        plsc.subcore_barrier()
        pltpu.sync_copy(acc.at[pl.ds(s * STRIPE, STRIPE)],
                        out_hbm.at[c].at[pl.ds(s * STRIPE, STRIPE)])

    return k(proc, src3, dst3, zeros_stripe)


def _sc_deg(dst3, zeros_deg, ones_row):
    @functools.partial(
        pl.kernel,
        out_type=jax.ShapeDtypeStruct((NC, DEG_PAD), jnp.float32),
        mesh=_vmesh(),
        scratch_types=[
            pltpu.VMEM((ROWS, B), jnp.int32),
            pltpu.VMEM((B,), jnp.float32),
            pltpu.VMEM_SHARED((DEG_PAD,), jnp.float32),
        ],
        compiler_params=_SC_PARAMS,
    )
    def k(dst_hbm, z_hbm, ones_hbm, out_hbm, dstv, onesv, acc):
        c = lax.axis_index("c")
        s = lax.axis_index("s")
        w = c * NS + s
        pltpu.sync_copy(z_hbm, acc.at[pl.ds(s * DEG_STRIPE, DEG_STRIPE)])
        pltpu.sync_copy(dst_hbm.at[w], dstv)
        pltpu.sync_copy(ones_hbm, onesv)
        plsc.subcore_barrier()

        def body(j, carry):
            pltpu.sync_copy(onesv, acc.at[dstv.at[j]], add=True)
            return carry

        lax.fori_loop(0, ROWS, body, 0)
        plsc.subcore_barrier()
        pltpu.sync_copy(acc.at[pl.ds(s * DEG_STRIPE, DEG_STRIPE)],
                        out_hbm.at[c].at[pl.ds(s * DEG_STRIPE, DEG_STRIPE)])

    return k(dst3, zeros_deg, ones_row)


def _lrelu(x):
    return jnp.where(x >= 0, x, 0.01 * x)


def _row2(v):
    return v.reshape(1, -1)


def _enc_call(x, p):
    def body(x_ref, w0, b0, w1, b1, w2, b2, g, bt, o_ref):
        h = _lrelu(x_ref[...] @ w0[...] + b0[...])
        h = _lrelu(h @ w1[...] + b1[...])
        h = h @ w2[...] + b2[...]
        mu = jnp.mean(h, axis=-1, keepdims=True)
        var = jnp.mean((h - mu) ** 2, axis=-1, keepdims=True)
        o_ref[...] = (h - mu) * lax.rsqrt(var + 1e-5) * g[...] + bt[...]

    return pl.pallas_call(
        body,
        out_shape=jax.ShapeDtypeStruct((N_NODES, LAT), jnp.float32),
    )(x, p['Win'], _row2(p['bin']), p['Wh'][0], _row2(p['bh'][0]),
      p['Wout'], _row2(p['bout']), _row2(p['gamma']), _row2(p['beta']))


def _proc_call(proc, S, deg, p):
    def body(x_ref, s_ref, d_ref, wa, wb, b0, w1, b1, w2, b2, g, bt, o_ref):
        x = x_ref[...]
        pe = (s_ref[0, :N_NODES, :] + s_ref[1, :N_NODES, :]
              + (d_ref[0, :N_NODES, :] + d_ref[1, :N_NODES, :]) * x)
        h = _lrelu(x @ wa[...] + pe @ wb[...] + b0[...])
        h = _lrelu(h @ w1[...] + b1[...])
        h = h @ w2[...] + b2[...]
        mu = jnp.mean(h, axis=-1, keepdims=True)
        var = jnp.mean((h - mu) ** 2, axis=-1, keepdims=True)
        o_ref[...] = (h - mu) * lax.rsqrt(var + 1e-5) * g[...] + bt[...] + x

    return pl.pallas_call(
        body,
        out_shape=jax.ShapeDtypeStruct((N_NODES, LAT), jnp.float32),
    )(proc, S, deg, p['Win'][:LAT], p['Win'][LAT:], _row2(p['bin']),
      p['Wh'][0], _row2(p['bh'][0]), p['Wout'], _row2(p['bout']),
      _row2(p['gamma']), _row2(p['beta']))


def _out_call(proc, p):
    def body(x_ref, w0, b0, w1, b1, w2, b2, o_ref):
        h = _lrelu(x_ref[...] @ w0[...] + b0[...])
        h = _lrelu(h @ w1[...] + b1[...])
        o_ref[...] = h @ w2[...] + b2[...]

    return pl.pallas_call(
        body,
        out_shape=jax.ShapeDtypeStruct((N_NODES, p['Wout'].shape[1]), jnp.float32),
    )(proc, p['Win'], _row2(p['bin']), p['Wh'][0], _row2(p['bh'][0]),
      p['Wout'], _row2(p['bout']))


def kernel(in_feat, edge_index, params):
    src = edge_index[0]
    dst = edge_index[1]
    npad = E_PAD - src.shape[0]
    src3 = jnp.concatenate([src, jnp.zeros((npad,), jnp.int32)]).reshape(NW, ROWS, B)
    dst3 = jnp.concatenate([dst, jnp.full((npad,), N_NODES, jnp.int32)]).reshape(NW, ROWS, B)
    zeros_s = jnp.zeros((STRIPE, LAT), jnp.float32)
    zeros_d = jnp.zeros((DEG_STRIPE,), jnp.float32)
    ones_r = jnp.ones((B,), jnp.float32)

    deg = _sc_deg(dst3, zeros_d, ones_r).reshape(NC, DEG_PAD, 1)
    proc = _enc_call(in_feat, params['enc'])
    for i in range(N_ITERS):
        S = _sc_agg(proc, src3, dst3, zeros_s)
        proc = _proc_call(proc, S, deg, params['proc'][i])
    return _out_call(proc, params['out'])


# speedup vs baseline: 13.2382x; 1.0113x over previous
import functools

import jax
import jax.numpy as jnp
from jax import lax
from jax.experimental import pallas as pl
from jax.experimental.pallas import tpu as pltpu
from jax.experimental.pallas import tpu_sc as plsc

N_NODES = 10000
LAT = 32
N_ITERS = 3
NC = 2
NS = 16
NW = NC * NS
B = 128
ROWS = 80
E_PAD = NW * ROWS * B
PAD_NODES = 10112
STRIPE = PAD_NODES // NS
DEG_PAD = 10240
DEG_STRIPE = DEG_PAD // NS


def _vmesh():
    return plsc.VectorSubcoreMesh(core_axis_name="c", subcore_axis_name="s")


_SC_PARAMS = pltpu.CompilerParams(use_tc_tiling_on_sc=False)


def _sc_agg(proc, src3, dst3, zeros_stripe):
    @functools.partial(
        pl.kernel,
        out_type=jax.ShapeDtypeStruct((NC, PAD_NODES, LAT), jnp.float32),
        mesh=_vmesh(),
        scratch_types=[
            pltpu.VMEM((ROWS, B), jnp.int32),
            pltpu.VMEM((ROWS, B), jnp.int32),
            pltpu.VMEM((B, LAT), jnp.float32),
            pltpu.VMEM((B, LAT), jnp.float32),
            pltpu.VMEM_SHARED((PAD_NODES, LAT), jnp.float32),
            pltpu.SemaphoreType.DMA,
            pltpu.SemaphoreType.DMA,
        ],
        compiler_params=_SC_PARAMS,
    )
    def k(proc_hbm, src_hbm, dst_hbm, z_hbm, out_hbm,
          srcv, dstv, rows0, rows1, acc, sem0, sem1):
        c = lax.axis_index("c")
        s = lax.axis_index("s")
        w = c * NS + s
        pltpu.sync_copy(z_hbm, acc.at[pl.ds(s * STRIPE, STRIPE)])
        pltpu.sync_copy(src_hbm.at[w], srcv)
        pltpu.sync_copy(dst_hbm.at[w], dstv)
        plsc.subcore_barrier()

        pltpu.async_copy(proc_hbm.at[srcv.at[0]], rows0, sem0)

        def body(jj, carry):
            j0 = 2 * jj
            pltpu.async_copy(proc_hbm.at[srcv.at[j0 + 1]], rows1, sem1)
            pltpu.make_async_copy(proc_hbm.at[srcv.at[j0]], rows0, sem0).wait()
            pltpu.sync_copy(rows0, acc.at[dstv.at[j0]], add=True)
            pltpu.async_copy(proc_hbm.at[srcv.at[j0 + 2]], rows0, sem0)
            pltpu.make_async_copy(proc_hbm.at[srcv.at[j0 + 1]], rows1, sem1).wait()
            pltpu.sync_copy(rows1, acc.at[dstv.at[j0 + 1]], add=True)
            return carry

        lax.fori_loop(0, ROWS // 2 - 1, body, 0)
        pltpu.async_copy(proc_hbm.at[srcv.at[ROWS - 1]], rows1, sem1)
        pltpu.make_async_copy(proc_hbm.at[srcv.at[ROWS - 2]], rows0, sem0).wait()
        pltpu.sync_copy(rows0, acc.at[dstv.at[ROWS - 2]], add=True)
        pltpu.make_async_copy(proc_hbm.at[srcv.at[ROWS - 1]], rows1, sem1).wait()
        pltpu.sync_copy(rows1, acc.at[dstv.at[ROWS - 1]], add=True)
        plsc.subcore_barrier()
        pltpu.sync_copy(acc.at[pl.ds(s * STRIPE, STRIPE)],
                        out_hbm.at[c].at[pl.ds(s * STRIPE, STRIPE)])

    return k(proc, src3, dst3, zeros_stripe)


def _sc_deg(dst3, zeros_deg, ones_row):
    @functools.partial(
        pl.kernel,
        out_type=jax.ShapeDtypeStruct((NC, DEG_PAD), jnp.float32),
        mesh=_vmesh(),
        scratch_types=[
            pltpu.VMEM((ROWS, B), jnp.int32),
            pltpu.VMEM((B,), jnp.float32),
            pltpu.VMEM_SHARED((DEG_PAD,), jnp.float32),
        ],
        compiler_params=_SC_PARAMS,
    )
    def k(dst_hbm, z_hbm, ones_hbm, out_hbm, dstv, onesv, acc):
        c = lax.axis_index("c")
        s = lax.axis_index("s")
        w = c * NS + s
        pltpu.sync_copy(z_hbm, acc.at[pl.ds(s * DEG_STRIPE, DEG_STRIPE)])
        pltpu.sync_copy(dst_hbm.at[w], dstv)
        pltpu.sync_copy(ones_hbm, onesv)
        plsc.subcore_barrier()

        def body(j, carry):
            pltpu.sync_copy(onesv, acc.at[dstv.at[j]], add=True)
            return carry

        lax.fori_loop(0, ROWS, body, 0)
        plsc.subcore_barrier()
        pltpu.sync_copy(acc.at[pl.ds(s * DEG_STRIPE, DEG_STRIPE)],
                        out_hbm.at[c].at[pl.ds(s * DEG_STRIPE, DEG_STRIPE)])

    return k(dst3, zeros_deg, ones_row)


def _lrelu(x):
    return jnp.where(x >= 0, x, 0.01 * x)


def _row2(v):
    return v.reshape(1, -1)


def _enc_call(x, p):
    def body(x_ref, w0, b0, w1, b1, w2, b2, g, bt, o_ref):
        h = _lrelu(x_ref[...] @ w0[...] + b0[...])
        h = _lrelu(h @ w1[...] + b1[...])
        h = h @ w2[...] + b2[...]
        mu = jnp.mean(h, axis=-1, keepdims=True)
        var = jnp.mean((h - mu) ** 2, axis=-1, keepdims=True)
        o_ref[...] = (h - mu) * lax.rsqrt(var + 1e-5) * g[...] + bt[...]

    return pl.pallas_call(
        body,
        out_shape=jax.ShapeDtypeStruct((N_NODES, LAT), jnp.float32),
    )(x, p['Win'], _row2(p['bin']), p['Wh'][0], _row2(p['bh'][0]),
      p['Wout'], _row2(p['bout']), _row2(p['gamma']), _row2(p['beta']))


def _proc_call(proc, S, deg, p):
    def body(x_ref, s_ref, d_ref, wa, wb, b0, w1, b1, w2, b2, g, bt, o_ref):
        x = x_ref[...]
        pe = (s_ref[0, :N_NODES, :] + s_ref[1, :N_NODES, :]
              + (d_ref[0, :N_NODES, :] + d_ref[1, :N_NODES, :]) * x)
        h = _lrelu(x @ wa[...] + pe @ wb[...] + b0[...])
        h = _lrelu(h @ w1[...] + b1[...])
        h = h @ w2[...] + b2[...]
        mu = jnp.mean(h, axis=-1, keepdims=True)
        var = jnp.mean((h - mu) ** 2, axis=-1, keepdims=True)
        o_ref[...] = (h - mu) * lax.rsqrt(var + 1e-5) * g[...] + bt[...] + x

    return pl.pallas_call(
        body,
        out_shape=jax.ShapeDtypeStruct((N_NODES, LAT), jnp.float32),
    )(proc, S, deg, p['Win'][:LAT], p['Win'][LAT:], _row2(p['bin']),
      p['Wh'][0], _row2(p['bh'][0]), p['Wout'], _row2(p['bout']),
      _row2(p['gamma']), _row2(p['beta']))


def _out_call(proc, p):
    def body(x_ref, w0, b0, w1, b1, w2, b2, o_ref):
        h = _lrelu(x_ref[...] @ w0[...] + b0[...])
        h = _lrelu(h @ w1[...] + b1[...])
        o_ref[...] = h @ w2[...] + b2[...]

    return pl.pallas_call(
        body,
        out_shape=jax.ShapeDtypeStruct((N_NODES, p['Wout'].shape[1]), jnp.float32),
    )(proc, p['Win'], _row2(p['bin']), p['Wh'][0], _row2(p['bh'][0]),
      p['Wout'], _row2(p['bout']))


def kernel(in_feat, edge_index, params):
    src = edge_index[0]
    dst = edge_index[1]
    npad = E_PAD - src.shape[0]
    pad_dst = N_NODES + (jnp.arange(npad, dtype=jnp.int32) % (PAD_NODES - N_NODES))
    src3 = jnp.concatenate([src, jnp.zeros((npad,), jnp.int32)]).reshape(NW, ROWS, B)
    dst3 = jnp.concatenate([dst, pad_dst]).reshape(NW, ROWS, B)
    zeros_s = jnp.zeros((STRIPE, LAT), jnp.float32)
    zeros_d = jnp.zeros((DEG_STRIPE,), jnp.float32)
    ones_r = jnp.ones((B,), jnp.float32)

    deg = _sc_deg(dst3, zeros_d, ones_r).reshape(NC, DEG_PAD, 1)
    proc = _enc_call(in_feat, params['enc'])
    for i in range(N_ITERS):
        S = _sc_agg(proc, src3, dst3, zeros_s)
        proc = _proc_call(proc, S, deg, params['proc'][i])
    return _out_call(proc, params['out'])


# speedup vs baseline: 21.1428x; 1.5971x over previous
import functools

import jax
import jax.numpy as jnp
from jax import lax
from jax.experimental import pallas as pl
from jax.experimental.pallas import tpu as pltpu
from jax.experimental.pallas import tpu_sc as plsc

N_NODES = 10000
LAT = 32
N_ITERS = 3
NC = 2
NS = 16
NW = NC * NS
B = 128
ROWS = 80
E_PAD = NW * ROWS * B
PAD_NODES = 10112
STRIPE = PAD_NODES // NS
DEG_PAD = 10240
DEG_STRIPE = DEG_PAD // NS


def _vmesh():
    return plsc.VectorSubcoreMesh(core_axis_name="c", subcore_axis_name="s")


_SC_PARAMS = pltpu.CompilerParams(use_tc_tiling_on_sc=False)


def _sc_agg(proc, src3, dst3, zeros_stripe):
    @functools.partial(
        pl.kernel,
        out_type=jax.ShapeDtypeStruct((NC, PAD_NODES, LAT), jnp.float32),
        mesh=_vmesh(),
        scratch_types=[
            pltpu.VMEM((ROWS, B), jnp.int32),
            pltpu.VMEM((ROWS, B), jnp.int32),
            pltpu.VMEM((B, LAT), jnp.float32),
            pltpu.VMEM((B, LAT), jnp.float32),
            pltpu.VMEM_SHARED((PAD_NODES, LAT), jnp.float32),
            pltpu.SemaphoreType.DMA,
            pltpu.SemaphoreType.DMA,
        ],
        compiler_params=_SC_PARAMS,
    )
    def k(proc_hbm, src_hbm, dst_hbm, z_hbm, out_hbm,
          srcv, dstv, rows0, rows1, acc, sem0, sem1):
        c = lax.axis_index("c")
        s = lax.axis_index("s")
        w = c * NS + s
        pltpu.sync_copy(z_hbm, acc.at[pl.ds(s * STRIPE, STRIPE)])
        pltpu.sync_copy(src_hbm.at[w], srcv)
        pltpu.sync_copy(dst_hbm.at[w], dstv)
        plsc.subcore_barrier()

        pltpu.async_copy(proc_hbm.at[srcv.at[0]], rows0, sem0)

        def body(jj, carry):
            j0 = 2 * jj
            pltpu.async_copy(proc_hbm.at[srcv.at[j0 + 1]], rows1, sem1)
            pltpu.make_async_copy(proc_hbm.at[srcv.at[j0]], rows0, sem0).wait()
            pltpu.sync_copy(rows0, acc.at[dstv.at[j0]], add=True)
            pltpu.async_copy(proc_hbm.at[srcv.at[j0 + 2]], rows0, sem0)
            pltpu.make_async_copy(proc_hbm.at[srcv.at[j0 + 1]], rows1, sem1).wait()
            pltpu.sync_copy(rows1, acc.at[dstv.at[j0 + 1]], add=True)
            return carry

        lax.fori_loop(0, ROWS // 2 - 1, body, 0)
        pltpu.async_copy(proc_hbm.at[srcv.at[ROWS - 1]], rows1, sem1)
        pltpu.make_async_copy(proc_hbm.at[srcv.at[ROWS - 2]], rows0, sem0).wait()
        pltpu.sync_copy(rows0, acc.at[dstv.at[ROWS - 2]], add=True)
        pltpu.make_async_copy(proc_hbm.at[srcv.at[ROWS - 1]], rows1, sem1).wait()
        pltpu.sync_copy(rows1, acc.at[dstv.at[ROWS - 1]], add=True)
        plsc.subcore_barrier()
        pltpu.sync_copy(acc.at[pl.ds(s * STRIPE, STRIPE)],
                        out_hbm.at[c].at[pl.ds(s * STRIPE, STRIPE)])

    return k(proc, src3, dst3, zeros_stripe)


def _sc_deg(dst3, zeros_deg, ones_row):
    @functools.partial(
        pl.kernel,
        out_type=jax.ShapeDtypeStruct((NC, DEG_PAD), jnp.float32),
        mesh=_vmesh(),
        scratch_types=[
            pltpu.VMEM((ROWS, B), jnp.int32),
            pltpu.VMEM((B,), jnp.float32),
            pltpu.VMEM_SHARED((DEG_PAD,), jnp.float32),
        ],
        compiler_params=_SC_PARAMS,
    )
    def k(dst_hbm, z_hbm, ones_hbm, out_hbm, dstv, onesv, acc):
        c = lax.axis_index("c")
        s = lax.axis_index("s")
        w = c * NS + s
        pltpu.sync_copy(z_hbm, acc.at[pl.ds(s * DEG_STRIPE, DEG_STRIPE)])
        pltpu.sync_copy(dst_hbm.at[w], dstv)
        pltpu.sync_copy(ones_hbm, onesv)
        plsc.subcore_barrier()

        def body(j, carry):
            pltpu.sync_copy(onesv, acc.at[dstv.at[j]], add=True)
            return carry

        lax.fori_loop(0, ROWS, body, 0)
        plsc.subcore_barrier()
        pltpu.sync_copy(acc.at[pl.ds(s * DEG_STRIPE, DEG_STRIPE)],
                        out_hbm.at[c].at[pl.ds(s * DEG_STRIPE, DEG_STRIPE)])

    return k(dst3, zeros_deg, ones_row)


def _lrelu(x):
    return jnp.where(x >= 0, x, 0.01 * x)


def _row2(v):
    return v.reshape(1, -1)


def _enc_call(x, p):
    def body(x_ref, w0, b0, w1, b1, w2, b2, g, bt, o_ref):
        h = _lrelu(x_ref[...] @ w0[...] + b0[...])
        h = _lrelu(h @ w1[...] + b1[...])
        h = h @ w2[...] + b2[...]
        mu = jnp.mean(h, axis=-1, keepdims=True)
        var = jnp.mean((h - mu) ** 2, axis=-1, keepdims=True)
        o_ref[...] = (h - mu) * lax.rsqrt(var + 1e-5) * g[...] + bt[...]

    return pl.pallas_call(
        body,
        out_shape=jax.ShapeDtypeStruct((N_NODES, LAT), jnp.float32),
    )(x, p['Win'], _row2(p['bin']), p['Wh'][0], _row2(p['bh'][0]),
      p['Wout'], _row2(p['bout']), _row2(p['gamma']), _row2(p['beta']))


def _proc_call(proc, S, deg, p):
    def body(x_ref, s_ref, d_ref, wa, wb, b0, w1, b1, w2, b2, g, bt, o_ref):
        x = x_ref[...]
        pe = (s_ref[0, :N_NODES, :] + s_ref[1, :N_NODES, :]
              + (d_ref[0, :N_NODES, :] + d_ref[1, :N_NODES, :]) * x)
        h = _lrelu(x @ wa[...] + pe @ wb[...] + b0[...])
        h = _lrelu(h @ w1[...] + b1[...])
        h = h @ w2[...] + b2[...]
        mu = jnp.mean(h, axis=-1, keepdims=True)
        var = jnp.mean((h - mu) ** 2, axis=-1, keepdims=True)
        o_ref[...] = (h - mu) * lax.rsqrt(var + 1e-5) * g[...] + bt[...] + x

    return pl.pallas_call(
        body,
        out_shape=jax.ShapeDtypeStruct((N_NODES, LAT), jnp.float32),
    )(proc, S, deg, p['Win'][:LAT], p['Win'][LAT:], _row2(p['bin']),
      p['Wh'][0], _row2(p['bh'][0]), p['Wout'], _row2(p['bout']),
      _row2(p['gamma']), _row2(p['beta']))


def _out_call(proc, p):
    def body(x_ref, w0, b0, w1, b1, w2, b2, o_ref):
        h = _lrelu(x_ref[...] @ w0[...] + b0[...])
        h = _lrelu(h @ w1[...] + b1[...])
        o_ref[...] = h @ w2[...] + b2[...]

    return pl.pallas_call(
        body,
        out_shape=jax.ShapeDtypeStruct((N_NODES, p['Wout'].shape[1]), jnp.float32),
    )(proc, p['Win'], _row2(p['bin']), p['Wh'][0], _row2(p['bh'][0]),
      p['Wout'], _row2(p['bout']))


def kernel(in_feat, edge_index, params):
    src = edge_index[0]
    dst = edge_index[1]
    n_edges = src.shape[0]
    per_w = n_edges // NW
    padw = ROWS * B - per_w
    pad_src = jnp.broadcast_to(jnp.arange(padw, dtype=jnp.int32)[None], (NW, padw))
    pad_dst = jnp.broadcast_to(
        N_NODES + (jnp.arange(padw, dtype=jnp.int32) % (PAD_NODES - N_NODES))[None],
        (NW, padw))
    src3 = jnp.concatenate([src.reshape(NW, per_w), pad_src], 1).reshape(NW, ROWS, B)
    dst3 = jnp.concatenate([dst.reshape(NW, per_w), pad_dst], 1).reshape(NW, ROWS, B)
    zeros_s = jnp.zeros((STRIPE, LAT), jnp.float32)
    zeros_d = jnp.zeros((DEG_STRIPE,), jnp.float32)
    ones_r = jnp.ones((B,), jnp.float32)

    deg = _sc_deg(dst3, zeros_d, ones_r).reshape(NC, DEG_PAD, 1)
    proc = _enc_call(in_feat, params['enc'])
    for i in range(N_ITERS):
        S = _sc_agg(proc, src3, dst3, zeros_s)
        proc = _proc_call(proc, S, deg, params['proc'][i])
    return _out_call(proc, params['out'])


# speedup vs baseline: 25.1085x; 1.1876x over previous
import functools

import jax
import jax.numpy as jnp
from jax import lax
from jax.experimental import pallas as pl
from jax.experimental.pallas import tpu as pltpu
from jax.experimental.pallas import tpu_sc as plsc

N_NODES = 10000
LAT = 32
N_ITERS = 3
NC = 2
NS = 16
NW = NC * NS
B = 128
ROWS = 80
E_PAD = NW * ROWS * B
PAD_NODES = 10112
STRIPE = PAD_NODES // NS
DEG_PAD = 10240
DEG_STRIPE = DEG_PAD // NS


def _vmesh():
    return plsc.VectorSubcoreMesh(core_axis_name="c", subcore_axis_name="s")


_SC_PARAMS = pltpu.CompilerParams(use_tc_tiling_on_sc=False)


def _sc_agg(proc, src3, dst3, zeros_stripe):
    @functools.partial(
        pl.kernel,
        out_type=jax.ShapeDtypeStruct((NC, PAD_NODES, LAT), jnp.float32),
        mesh=_vmesh(),
        scratch_types=[
            pltpu.VMEM((ROWS, B), jnp.int32),
            pltpu.VMEM((ROWS, B), jnp.int32),
            pltpu.VMEM((B, LAT), jnp.float32),
            pltpu.VMEM((B, LAT), jnp.float32),
            pltpu.VMEM((B, LAT), jnp.float32),
            pltpu.VMEM((B, LAT), jnp.float32),
            pltpu.VMEM_SHARED((PAD_NODES, LAT), jnp.float32),
            pltpu.SemaphoreType.DMA,
            pltpu.SemaphoreType.DMA,
            pltpu.SemaphoreType.DMA,
            pltpu.SemaphoreType.DMA,
        ],
        compiler_params=_SC_PARAMS,
    )
    def k(proc_hbm, src_hbm, dst_hbm, z_hbm, out_hbm,
          srcv, dstv, rows0, rows1, rows2, rows3, acc, sem0, sem1, sem2, sem3):
        c = lax.axis_index("c")
        s = lax.axis_index("s")
        w = c * NS + s
        pltpu.sync_copy(z_hbm, acc.at[pl.ds(s * STRIPE, STRIPE)])
        pltpu.sync_copy(src_hbm.at[w], srcv)
        pltpu.sync_copy(dst_hbm.at[w], dstv)
        plsc.subcore_barrier()

        bufs = [(rows0, sem0), (rows1, sem1), (rows2, sem2), (rows3, sem3)]
        nb = len(bufs)

        def start(j, b):
            pltpu.async_copy(proc_hbm.at[srcv.at[j]], bufs[b][0], bufs[b][1])

        def drain(j, b):
            pltpu.make_async_copy(proc_hbm.at[srcv.at[j]],
                                  bufs[b][0], bufs[b][1]).wait()
            pltpu.sync_copy(bufs[b][0], acc.at[dstv.at[j]], add=True)

        for b in range(nb - 1):
            start(b, b)

        def body(jj, carry):
            j = nb * jj
            for b in range(nb):
                start(j + b + nb - 1, (b + nb - 1) % nb)
                drain(j + b, b)
            return carry

        lax.fori_loop(0, ROWS // nb - 1, body, 0)
        j = ROWS - nb
        start(ROWS - 1, nb - 1)
        for b in range(nb):
            drain(j + b, b)
        plsc.subcore_barrier()
        pltpu.sync_copy(acc.at[pl.ds(s * STRIPE, STRIPE)],
                        out_hbm.at[c].at[pl.ds(s * STRIPE, STRIPE)])

    return k(proc, src3, dst3, zeros_stripe)


def _sc_deg(dst3, zeros_deg, ones_row):
    @functools.partial(
        pl.kernel,
        out_type=jax.ShapeDtypeStruct((NC, DEG_PAD), jnp.float32),
        mesh=_vmesh(),
        scratch_types=[
            pltpu.VMEM((ROWS, B), jnp.int32),
            pltpu.VMEM((B,), jnp.float32),
            pltpu.VMEM_SHARED((DEG_PAD,), jnp.float32),
        ],
        compiler_params=_SC_PARAMS,
    )
    def k(dst_hbm, z_hbm, ones_hbm, out_hbm, dstv, onesv, acc):
        c = lax.axis_index("c")
        s = lax.axis_index("s")
        w = c * NS + s
        pltpu.sync_copy(z_hbm, acc.at[pl.ds(s * DEG_STRIPE, DEG_STRIPE)])
        pltpu.sync_copy(dst_hbm.at[w], dstv)
        pltpu.sync_copy(ones_hbm, onesv)
        plsc.subcore_barrier()

        def body(j, carry):
            pltpu.sync_copy(onesv, acc.at[dstv.at[j]], add=True)
            return carry

        lax.fori_loop(0, ROWS, body, 0)
        plsc.subcore_barrier()
        pltpu.sync_copy(acc.at[pl.ds(s * DEG_STRIPE, DEG_STRIPE)],
                        out_hbm.at[c].at[pl.ds(s * DEG_STRIPE, DEG_STRIPE)])

    return k(dst3, zeros_deg, ones_row)


def _lrelu(x):
    return jnp.where(x >= 0, x, 0.01 * x)


def _row2(v):
    return v.reshape(1, -1)


def _enc_call(x, p):
    def body(x_ref, w0, b0, w1, b1, w2, b2, g, bt, o_ref):
        h = _lrelu(x_ref[...] @ w0[...] + b0[...])
        h = _lrelu(h @ w1[...] + b1[...])
        h = h @ w2[...] + b2[...]
        mu = jnp.mean(h, axis=-1, keepdims=True)
        var = jnp.mean((h - mu) ** 2, axis=-1, keepdims=True)
        o_ref[...] = (h - mu) * lax.rsqrt(var + 1e-5) * g[...] + bt[...]

    return pl.pallas_call(
        body,
        out_shape=jax.ShapeDtypeStruct((N_NODES, LAT), jnp.float32),
    )(x, p['Win'], _row2(p['bin']), p['Wh'][0], _row2(p['bh'][0]),
      p['Wout'], _row2(p['bout']), _row2(p['gamma']), _row2(p['beta']))


def _proc_call(proc, S, deg, p):
    def body(x_ref, s_ref, d_ref, wa, wb, b0, w1, b1, w2, b2, g, bt, o_ref):
        x = x_ref[...]
        pe = (s_ref[0, :N_NODES, :] + s_ref[1, :N_NODES, :]
              + (d_ref[0, :N_NODES, :] + d_ref[1, :N_NODES, :]) * x)
        h = _lrelu(x @ wa[...] + pe @ wb[...] + b0[...])
        h = _lrelu(h @ w1[...] + b1[...])
        h = h @ w2[...] + b2[...]
        mu = jnp.mean(h, axis=-1, keepdims=True)
        var = jnp.mean((h - mu) ** 2, axis=-1, keepdims=True)
        o_ref[...] = (h - mu) * lax.rsqrt(var + 1e-5) * g[...] + bt[...] + x

    return pl.pallas_call(
        body,
        out_shape=jax.ShapeDtypeStruct((N_NODES, LAT), jnp.float32),
    )(proc, S, deg, p['Win'][:LAT], p['Win'][LAT:], _row2(p['bin']),
      p['Wh'][0], _row2(p['bh'][0]), p['Wout'], _row2(p['bout']),
      _row2(p['gamma']), _row2(p['beta']))


def _out_call(proc, p):
    def body(x_ref, w0, b0, w1, b1, w2, b2, o_ref):
        h = _lrelu(x_ref[...] @ w0[...] + b0[...])
        h = _lrelu(h @ w1[...] + b1[...])
        o_ref[...] = h @ w2[...] + b2[...]

    return pl.pallas_call(
        body,
        out_shape=jax.ShapeDtypeStruct((N_NODES, p['Wout'].shape[1]), jnp.float32),
    )(proc, p['Win'], _row2(p['bin']), p['Wh'][0], _row2(p['bh'][0]),
      p['Wout'], _row2(p['bout']))


def kernel(in_feat, edge_index, params):
    src = edge_index[0]
    dst = edge_index[1]
    n_edges = src.shape[0]
    per_w = n_edges // NW
    padw = ROWS * B - per_w
    pad_src = jnp.broadcast_to(jnp.arange(padw, dtype=jnp.int32)[None], (NW, padw))
    pad_dst = jnp.broadcast_to(
        N_NODES + (jnp.arange(padw, dtype=jnp.int32) % (PAD_NODES - N_NODES))[None],
        (NW, padw))
    src3 = jnp.concatenate([src.reshape(NW, per_w), pad_src], 1).reshape(NW, ROWS, B)
    dst3 = jnp.concatenate([dst.reshape(NW, per_w), pad_dst], 1).reshape(NW, ROWS, B)
    zeros_s = jnp.zeros((STRIPE, LAT), jnp.float32)
    zeros_d = jnp.zeros((DEG_STRIPE,), jnp.float32)
    ones_r = jnp.ones((B,), jnp.float32)

    deg = _sc_deg(dst3, zeros_d, ones_r).reshape(NC, DEG_PAD, 1)
    proc = _enc_call(in_feat, params['enc'])
    for i in range(N_ITERS):
        S = _sc_agg(proc, src3, dst3, zeros_s)
        proc = _proc_call(proc, S, deg, params['proc'][i])
    return _out_call(proc, params['out'])
